# baseline (speedup 1.0000x reference)
import functools

import jax
import jax.numpy as jnp
from jax import lax
from jax.experimental import pallas as pl
from jax.experimental.pallas import tpu as pltpu
from jax.experimental.pallas import tpu_sc as plsc

N_NODES = 100000
N_EDGES = 6400000
NC = 2
NS = 16
L = 16
N_PAD = 100096
RPT = N_PAD // NS
CHUNK = 4000
EDGES_PER_TILE = N_EDGES // (NC * NS)
CHUNKS = EDGES_PER_TILE // CHUNK
GROUPS = CHUNK // L


def _sc_edge_pass(edge_src, edge_dst, edge_g, edge_b, pred_e, pred_f,
                  zeros_tile, probe, params):
    mesh = plsc.VectorSubcoreMesh(
        core_axis_name="c", subcore_axis_name="s", num_cores=NC, num_subcores=NS
    )
    acc_ty = jax.ShapeDtypeStruct((NC * N_PAD,), jnp.float32)

    def buf(shape, dtype=jnp.float32):
        return pltpu.VMEM(shape, dtype)

    scratch = dict(
        stage_v=buf((RPT,)),
        flag_v=buf((L,), jnp.int32),
        probe_v=buf((2, L), jnp.int32),
        par_v=buf((4, L)),
        table_e=pltpu.VMEM_SHARED((N_PAD,), jnp.float32),
        table_f=pltpu.VMEM_SHARED((N_PAD,), jnp.float32),
        accfP=pltpu.VMEM_SHARED((N_PAD,), jnp.float32),
        accfQ=pltpu.VMEM_SHARED((N_PAD,), jnp.float32),
        accrP=pltpu.VMEM_SHARED((N_PAD,), jnp.float32),
        accrQ=pltpu.VMEM_SHARED((N_PAD,), jnp.float32),
        mfP_v=buf((CHUNK,)),
        mfQ_v=buf((CHUNK,)),
        mrP_v=buf((CHUNK,)),
        mrQ_v=buf((CHUNK,)),
        ssem0=pltpu.SemaphoreType.DMA,
        ssem1=pltpu.SemaphoreType.DMA,
        ssem2=pltpu.SemaphoreType.DMA,
        ssem3=pltpu.SemaphoreType.DMA,
    )
    for b in (0, 1):
        scratch.update({
            f"src_v{b}": buf((CHUNK,), jnp.int32),
            f"dst_v{b}": buf((CHUNK,), jnp.int32),
            f"gv_v{b}": buf((CHUNK,)),
            f"bv_v{b}": buf((CHUNK,)),
            f"es_v{b}": buf((CHUNK,)),
            f"fs_v{b}": buf((CHUNK,)),
            f"ed_v{b}": buf((CHUNK,)),
            f"fd_v{b}": buf((CHUNK,)),
            f"insem{b}": pltpu.SemaphoreType.DMA,
            f"gsem0{b}": pltpu.SemaphoreType.DMA,
            f"gsem1{b}": pltpu.SemaphoreType.DMA,
            f"gsem2{b}": pltpu.SemaphoreType.DMA,
            f"gsem3{b}": pltpu.SemaphoreType.DMA,
        })

    @functools.partial(
        pl.kernel,
        mesh=mesh,
        compiler_params=pltpu.CompilerParams(needs_layout_passes=False),
        out_type=(acc_ty, acc_ty, acc_ty, acc_ty,
                  jax.ShapeDtypeStruct((NC * NS * L,), jnp.int32)),
        scratch_types=scratch,
    )
    def body(es_hbm, ed_hbm, eg_hbm, eb_hbm, pe_hbm, pf_hbm, z_hbm, probe_hbm, par_hbm,
             outfP_hbm, outfQ_hbm, outrP_hbm, outrQ_hbm, flags_hbm,
             **refs):
        c = lax.axis_index("c")
        s = lax.axis_index("s")
        row0 = s * RPT
        stage_v = refs["stage_v"]
        flag_v = refs["flag_v"]
        probe_v = refs["probe_v"]
        par_v = refs["par_v"]
        table_e = refs["table_e"]
        table_f = refs["table_f"]
        accs = (refs["accfP"], refs["accfQ"], refs["accrP"], refs["accrQ"])
        msgs = (refs["mfP_v"], refs["mfQ_v"], refs["mrP_v"], refs["mrQ_v"])
        ssems = (refs["ssem0"], refs["ssem1"], refs["ssem2"], refs["ssem3"])
        ins = [(refs[f"src_v{b}"], refs[f"dst_v{b}"], refs[f"gv_v{b}"],
                refs[f"bv_v{b}"], refs[f"insem{b}"]) for b in (0, 1)]
        gbufs = [(refs[f"es_v{b}"], refs[f"fs_v{b}"], refs[f"ed_v{b}"],
                  refs[f"fd_v{b}"],
                  (refs[f"gsem0{b}"], refs[f"gsem1{b}"],
                   refs[f"gsem2{b}"], refs[f"gsem3{b}"])) for b in (0, 1)]

        pltpu.sync_copy(pe_hbm.at[pl.ds(row0, RPT)], stage_v)
        pltpu.sync_copy(stage_v, table_e.at[pl.ds(row0, RPT)])
        pltpu.sync_copy(pf_hbm.at[pl.ds(row0, RPT)], stage_v)
        pltpu.sync_copy(stage_v, table_f.at[pl.ds(row0, RPT)])
        pltpu.sync_copy(z_hbm, stage_v)
        for acc in accs:
            pltpu.sync_copy(stage_v, acc.at[pl.ds(row0, RPT)])
        pltpu.sync_copy(probe_hbm, probe_v)
        pltpu.sync_copy(par_hbm, par_v)
        flag_v[...] = jnp.zeros((L,), jnp.int32)
        plsc.subcore_barrier()

        src0 = probe_v[0, :]
        dst0 = probe_v[1, :]
        esg = par_v[0, :]
        esb = par_v[1, :]
        emg = par_v[2, :]
        emb = par_v[3, :]
        tile_base = (c * NS + s) * EDGES_PER_TILE

        def issue_in(t, b):
            base = tile_base + t * CHUNK
            src_v, dst_v, gv_v, bv_v, insem = ins[b]
            pltpu.async_copy(es_hbm.at[pl.ds(base, CHUNK)], src_v, insem)
            pltpu.async_copy(ed_hbm.at[pl.ds(base, CHUNK)], dst_v, insem)
            pltpu.async_copy(eg_hbm.at[pl.ds(base, CHUNK)], gv_v, insem)
            pltpu.async_copy(eb_hbm.at[pl.ds(base, CHUNK)], bv_v, insem)

        def wait_in(t, b):
            base = tile_base + t * CHUNK
            src_v, dst_v, gv_v, bv_v, insem = ins[b]
            pltpu.make_async_copy(es_hbm.at[pl.ds(base, CHUNK)], src_v, insem).wait()
            pltpu.make_async_copy(ed_hbm.at[pl.ds(base, CHUNK)], dst_v, insem).wait()
            pltpu.make_async_copy(eg_hbm.at[pl.ds(base, CHUNK)], gv_v, insem).wait()
            pltpu.make_async_copy(eb_hbm.at[pl.ds(base, CHUNK)], bv_v, insem).wait()

        def issue_gathers(b):
            src_v, dst_v, _, _, _ = ins[b]
            es_v, fs_v, ed_v, fd_v, gsems = gbufs[b]
            pltpu.async_copy(table_e.at[src_v], es_v, gsems[0])
            pltpu.async_copy(table_f.at[src_v], fs_v, gsems[1])
            pltpu.async_copy(table_e.at[dst_v], ed_v, gsems[2])
            pltpu.async_copy(table_f.at[dst_v], fd_v, gsems[3])

        def wait_gathers(b):
            src_v, dst_v, _, _, _ = ins[b]
            es_v, fs_v, ed_v, fd_v, gsems = gbufs[b]
            pltpu.make_async_copy(table_e.at[src_v], es_v, gsems[0]).wait()
            pltpu.make_async_copy(table_f.at[src_v], fs_v, gsems[1]).wait()
            pltpu.make_async_copy(table_e.at[dst_v], ed_v, gsems[2]).wait()
            pltpu.make_async_copy(table_f.at[dst_v], fd_v, gsems[3]).wait()

        def compute(b):
            src_v, dst_v, gv_v, bv_v, _ = ins[b]
            es_v, fs_v, ed_v, fd_v, _ = gbufs[b]
            mfP_v, mfQ_v, mrP_v, mrQ_v = msgs

            def group_body(g, _):
                sl = pl.ds(g * L, L)
                srcv = src_v[sl]
                dstv = dst_v[sl]
                e_i = es_v[sl]
                f_i = fs_v[sl]
                e_j = ed_v[sl]
                f_j = fd_v[sl]
                g_ij = gv_v[sl] * esg + emg
                b_ij = bv_v[sl] * esb + emb
                term1 = e_i * e_j + f_i * f_j
                term2 = f_i * e_j - e_i * f_j
                vt_f = (e_i * e_i + f_i * f_i) - term1
                vt_r = (e_j * e_j + f_j * f_j) - term1
                gt2 = g_ij * term2
                bt2 = b_ij * term2
                mfP_v[sl] = g_ij * vt_f - bt2
                mfQ_v[sl] = -(b_ij * vt_f) - gt2
                mrP_v[sl] = g_ij * vt_r + bt2
                mrQ_v[sl] = -(b_ij * vt_r) + gt2
                m = (srcv == dst0) & (dstv == src0)
                flag_v[...] = flag_v[...] | m.astype(jnp.int32)
                return 0

            lax.fori_loop(0, GROUPS, group_body, 0, unroll=2)

        def issue_scatter(b):
            src_v, dst_v, _, _, _ = ins[b]
            idxs = (src_v, src_v, dst_v, dst_v)
            for m, a, i, sm in zip(msgs, accs, idxs, ssems):
                pltpu.async_copy(m, a.at[i], sm, add=True)

        def wait_scatter(b):
            src_v, dst_v, _, _, _ = ins[b]
            idxs = (src_v, src_v, dst_v, dst_v)
            for m, a, i, sm in zip(msgs, accs, idxs, ssems):
                pltpu.make_async_copy(m, a.at[i], sm).wait()

        issue_in(0, 0)
        issue_in(1, 1)
        wait_in(0, 0)
        issue_gathers(0)

        def outer(o, carry):
            for b in (0, 1):
                t = o * 2 + b
                nb = 1 - b
                wait_gathers(b)
                compute(b)
                issue_scatter(b)

                @pl.when(t + 1 < CHUNKS)
                def _():
                    wait_in(t + 1, nb)
                    issue_gathers(nb)

                wait_scatter(b)

                @pl.when(t + 2 < CHUNKS)
                def _():
                    issue_in(t + 2, b)
            return carry

        lax.fori_loop(0, CHUNKS // 2, outer, 0, unroll=False)
        plsc.subcore_barrier()

        out0 = c * N_PAD + row0
        for acc, out in zip(accs, (outfP_hbm, outfQ_hbm, outrP_hbm, outrQ_hbm)):
            pltpu.sync_copy(acc.at[pl.ds(row0, RPT)], stage_v)
            pltpu.sync_copy(stage_v, out.at[pl.ds(out0, RPT)])
        pltpu.sync_copy(flag_v, flags_hbm.at[pl.ds((c * NS + s) * L, L)])

    return body(edge_src, edge_dst, edge_g, edge_b, pred_e, pred_f,
                zeros_tile, probe, params)


def _tc_losses(planes, xymean, xystd):
    def body(pe_e, pe_f, ty_p, ty_q, ty_e, ty_f, mk_e, mk_f,
             af0P, af0Q, af1P, af1Q, ar0P, ar0Q, ar1P, ar1Q, bt, tvm,
             xym, xys,
             mse_num, mse_den, a_out, b_out, pv_num, pv_den):
        xys_v = xys[...]
        xym_v = xym[...]
        mk_e_v = mk_e[...]
        mk_f_v = mk_f[...]
        sq = (pe_e[...] - ty_e[...]) ** 2 * mk_e_v + (pe_f[...] - ty_f[...]) ** 2 * mk_f_v
        mse_num[...] = jnp.sum(sq).reshape(1, 1)
        mse_den[...] = (jnp.sum(mk_e_v) + jnp.sum(mk_f_v)).reshape(1, 1)

        u_p = ty_p[...] * (xys_v[0:1, 0:1] + 1e-7) + xym_v[0:1, 0:1] + af0P[...] + af1P[...]
        u_q = ty_q[...] * (xys_v[0:1, 1:2] + 1e-7) + xym_v[0:1, 1:2] + af0Q[...] + af1Q[...]
        v_p = ar0P[...] + ar1P[...]
        v_q = ar0Q[...] + ar1Q[...]
        a_out[...] = (jnp.sum(u_p * u_p) + jnp.sum(u_q * u_q)).reshape(1, 1)
        b_out[...] = (2.0 * (jnp.sum(u_p * v_p) + jnp.sum(u_q * v_q))
                      + jnp.sum(v_p * v_p) + jnp.sum(v_q * v_q)).reshape(1, 1)

        er = pe_e[...] * (xys_v[0:1, 2:3] + 1e-7) + xym_v[0:1, 2:3]
        fr = pe_f[...] * (xys_v[0:1, 3:4] + 1e-7) + xym_v[0:1, 3:4]
        vm_sq = er * er + fr * fr
        tv = tvm[...]
        ispv = (bt[...] == 1).astype(jnp.float32)
        pv_num[...] = jnp.sum(jnp.abs(vm_sq - tv * tv) * ispv).reshape(1, 1)
        pv_den[...] = jnp.sum(ispv).reshape(1, 1)

    scalar = jax.ShapeDtypeStruct((1, 1), jnp.float32)
    return pl.pallas_call(
        body,
        out_shape=(scalar,) * 6,
    )(*planes, xymean, xystd)


def kernel(pred_ef, target_y, mask, edge_index, edge_attr, bus_type, target_vm,
           xymean, xystd, edgemean, edgestd):
    pred_e = jnp.pad(pred_ef[:, 0], (0, N_PAD - N_NODES))
    pred_f = jnp.pad(pred_ef[:, 1], (0, N_PAD - N_NODES))
    zeros_tile = jnp.zeros((RPT,), jnp.float32)
    probe = jnp.broadcast_to(edge_index[:, 0:1], (2, L)).astype(jnp.int32)
    params = jnp.broadcast_to(
        jnp.stack([edgestd[0, 0] + 1e-7, edgestd[0, 1] + 1e-7,
                   edgemean[0, 0], edgemean[0, 1]])[:, None], (4, L))

    accfP, accfQ, accrP, accrQ, flags = _sc_edge_pass(
        edge_index[0].astype(jnp.int32), edge_index[1].astype(jnp.int32),
        edge_attr[:, 0], edge_attr[:, 1], pred_e, pred_f, zeros_tile, probe,
        params)

    def plane(x):
        return x.reshape(80, 1250)

    def halves(acc):
        return plane(acc[:N_NODES]), plane(acc[N_PAD:N_PAD + N_NODES])

    af0P, af1P = halves(accfP)
    af0Q, af1Q = halves(accfQ)
    ar0P, ar1P = halves(accrP)
    ar0Q, ar1Q = halves(accrQ)

    planes = (
        plane(pred_ef[:, 0]), plane(pred_ef[:, 1]),
        plane(target_y[:, 0]), plane(target_y[:, 1]),
        plane(target_y[:, 2]), plane(target_y[:, 3]),
        plane(mask[:, 2]), plane(mask[:, 3]),
        af0P, af0Q, af1P, af1Q, ar0P, ar0Q, ar1P, ar1Q,
        plane(bus_type.astype(jnp.int32)), plane(target_vm),
    )
    mse_num, mse_den, a_sum, b_sum, pv_num, pv_den = _tc_losses(
        planes, xymean, xystd)

    has_rev = jnp.any(flags != 0)
    w_rev = jnp.where(has_rev, 1.0, 0.0).astype(jnp.float32)

    loss_mse = mse_num[0, 0] / (mse_den[0, 0] + 1e-6)
    loss_phys = (a_sum[0, 0] + (1.0 - w_rev) * b_sum[0, 0]) / jnp.float32(N_NODES * 2)
    n_pv = pv_den[0, 0]
    loss_pv = jnp.where(n_pv > 0,
                        pv_num[0, 0] / jnp.maximum(n_pv, 1.0),
                        jnp.float32(0.0))
    total = 0.8 * loss_mse + 0.2 * loss_phys + 0.1 * loss_pv
    return (total, loss_mse, loss_phys, loss_pv)


# speedup vs baseline: 1.5626x; 1.5626x over previous
import functools

import jax
import jax.numpy as jnp
from jax import lax
from jax.experimental import pallas as pl
from jax.experimental.pallas import tpu as pltpu
from jax.experimental.pallas import tpu_sc as plsc

N_NODES = 100000
N_EDGES = 6400000
NC = 2
NS = 16
L = 16
N_PAD = 100096
RPT = N_PAD // NS
CHUNK = 4000
EDGES_PER_TILE = N_EDGES // (NC * NS)
CHUNKS = EDGES_PER_TILE // CHUNK
GROUPS = CHUNK // L


def _sc_edge_pass(edge_src, edge_dst, edge_g, edge_b, pred_e, pred_f,
                  zeros_tile, probe, params):
    mesh = plsc.VectorSubcoreMesh(
        core_axis_name="c", subcore_axis_name="s", num_cores=NC, num_subcores=NS
    )
    acc_ty = jax.ShapeDtypeStruct((NC * N_PAD,), jnp.float32)

    def buf(shape, dtype=jnp.float32):
        return pltpu.VMEM(shape, dtype)

    scratch = dict(
        stage_v=buf((RPT,)),
        flag_v=buf((L,), jnp.int32),
        probe_v=buf((2, L), jnp.int32),
        par_v=buf((4, L)),
        table_e=pltpu.VMEM_SHARED((N_PAD,), jnp.float32),
        table_f=pltpu.VMEM_SHARED((N_PAD,), jnp.float32),
        accfP=pltpu.VMEM_SHARED((N_PAD,), jnp.float32),
        accfQ=pltpu.VMEM_SHARED((N_PAD,), jnp.float32),
        accrP=pltpu.VMEM_SHARED((N_PAD,), jnp.float32),
        accrQ=pltpu.VMEM_SHARED((N_PAD,), jnp.float32),
        mfP_v=buf((CHUNK,)),
        mfQ_v=buf((CHUNK,)),
        mrP_v=buf((CHUNK,)),
        mrQ_v=buf((CHUNK,)),
        ssem0=pltpu.SemaphoreType.DMA,
        ssem1=pltpu.SemaphoreType.DMA,
        ssem2=pltpu.SemaphoreType.DMA,
        ssem3=pltpu.SemaphoreType.DMA,
    )
    for b in (0, 1):
        scratch.update({
            f"src_v{b}": buf((CHUNK,), jnp.int32),
            f"dst_v{b}": buf((CHUNK,), jnp.int32),
            f"gv_v{b}": buf((CHUNK,)),
            f"bv_v{b}": buf((CHUNK,)),
            f"es_v{b}": buf((CHUNK,)),
            f"fs_v{b}": buf((CHUNK,)),
            f"ed_v{b}": buf((CHUNK,)),
            f"fd_v{b}": buf((CHUNK,)),
            f"insem{b}": pltpu.SemaphoreType.DMA,
            f"gsem0{b}": pltpu.SemaphoreType.DMA,
            f"gsem1{b}": pltpu.SemaphoreType.DMA,
            f"gsem2{b}": pltpu.SemaphoreType.DMA,
            f"gsem3{b}": pltpu.SemaphoreType.DMA,
        })

    @functools.partial(
        pl.kernel,
        mesh=mesh,
        compiler_params=pltpu.CompilerParams(needs_layout_passes=False),
        out_type=(acc_ty, acc_ty, acc_ty, acc_ty,
                  jax.ShapeDtypeStruct((NC * NS * L,), jnp.int32)),
        scratch_types=scratch,
    )
    def body(es_hbm, ed_hbm, eg_hbm, eb_hbm, pe_hbm, pf_hbm, z_hbm, probe_hbm, par_hbm,
             outfP_hbm, outfQ_hbm, outrP_hbm, outrQ_hbm, flags_hbm,
             **refs):
        c = lax.axis_index("c")
        s = lax.axis_index("s")
        row0 = s * RPT
        stage_v = refs["stage_v"]
        flag_v = refs["flag_v"]
        probe_v = refs["probe_v"]
        par_v = refs["par_v"]
        table_e = refs["table_e"]
        table_f = refs["table_f"]
        accs = (refs["accfP"], refs["accfQ"], refs["accrP"], refs["accrQ"])
        msgs = (refs["mfP_v"], refs["mfQ_v"], refs["mrP_v"], refs["mrQ_v"])
        ssems = (refs["ssem0"], refs["ssem1"], refs["ssem2"], refs["ssem3"])
        ins = [(refs[f"src_v{b}"], refs[f"dst_v{b}"], refs[f"gv_v{b}"],
                refs[f"bv_v{b}"], refs[f"insem{b}"]) for b in (0, 1)]
        gbufs = [(refs[f"es_v{b}"], refs[f"fs_v{b}"], refs[f"ed_v{b}"],
                  refs[f"fd_v{b}"],
                  (refs[f"gsem0{b}"], refs[f"gsem1{b}"],
                   refs[f"gsem2{b}"], refs[f"gsem3{b}"])) for b in (0, 1)]

        pltpu.sync_copy(pe_hbm.at[pl.ds(row0, RPT)], stage_v)
        pltpu.sync_copy(stage_v, table_e.at[pl.ds(row0, RPT)])
        pltpu.sync_copy(pf_hbm.at[pl.ds(row0, RPT)], stage_v)
        pltpu.sync_copy(stage_v, table_f.at[pl.ds(row0, RPT)])
        pltpu.sync_copy(z_hbm, stage_v)
        for acc in accs:
            pltpu.sync_copy(stage_v, acc.at[pl.ds(row0, RPT)])
        pltpu.sync_copy(probe_hbm, probe_v)
        pltpu.sync_copy(par_hbm, par_v)
        flag_v[...] = jnp.zeros((L,), jnp.int32)
        plsc.subcore_barrier()

        src0 = probe_v[0, :]
        dst0 = probe_v[1, :]
        esg = par_v[0, :]
        esb = par_v[1, :]
        emg = par_v[2, :]
        emb = par_v[3, :]
        tile_base = (c * NS + s) * EDGES_PER_TILE

        def issue_in(t, b):
            base = tile_base + t * CHUNK
            src_v, dst_v, gv_v, bv_v, insem = ins[b]
            pltpu.async_copy(es_hbm.at[pl.ds(base, CHUNK)], src_v, insem)
            pltpu.async_copy(ed_hbm.at[pl.ds(base, CHUNK)], dst_v, insem)
            pltpu.async_copy(eg_hbm.at[pl.ds(base, CHUNK)], gv_v, insem)
            pltpu.async_copy(eb_hbm.at[pl.ds(base, CHUNK)], bv_v, insem)

        def wait_in(t, b):
            base = tile_base + t * CHUNK
            src_v, dst_v, gv_v, bv_v, insem = ins[b]
            pltpu.make_async_copy(es_hbm.at[pl.ds(base, CHUNK)], src_v, insem).wait()
            pltpu.make_async_copy(ed_hbm.at[pl.ds(base, CHUNK)], dst_v, insem).wait()
            pltpu.make_async_copy(eg_hbm.at[pl.ds(base, CHUNK)], gv_v, insem).wait()
            pltpu.make_async_copy(eb_hbm.at[pl.ds(base, CHUNK)], bv_v, insem).wait()

        def issue_gathers(b):
            src_v, dst_v, _, _, _ = ins[b]
            es_v, fs_v, ed_v, fd_v, gsems = gbufs[b]
            pltpu.async_copy(table_e.at[src_v], es_v, gsems[0])
            pltpu.async_copy(table_f.at[src_v], fs_v, gsems[1])
            pltpu.async_copy(table_e.at[dst_v], ed_v, gsems[2])
            pltpu.async_copy(table_f.at[dst_v], fd_v, gsems[3])

        def wait_gathers(b):
            src_v, dst_v, _, _, _ = ins[b]
            es_v, fs_v, ed_v, fd_v, gsems = gbufs[b]
            pltpu.make_async_copy(table_e.at[src_v], es_v, gsems[0]).wait()
            pltpu.make_async_copy(table_f.at[src_v], fs_v, gsems[1]).wait()
            pltpu.make_async_copy(table_e.at[dst_v], ed_v, gsems[2]).wait()
            pltpu.make_async_copy(table_f.at[dst_v], fd_v, gsems[3]).wait()

        def compute(b):
            src_v, dst_v, gv_v, bv_v, _ = ins[b]
            es_v, fs_v, ed_v, fd_v, _ = gbufs[b]
            mfP_v, mfQ_v, mrP_v, mrQ_v = msgs

            def group_body(g, _):
                sl = pl.ds(g * L, L)
                srcv = src_v[sl]
                dstv = dst_v[sl]
                e_i = es_v[sl]
                f_i = fs_v[sl]
                e_j = ed_v[sl]
                f_j = fd_v[sl]
                g_ij = gv_v[sl] * esg + emg
                b_ij = bv_v[sl] * esb + emb
                term1 = e_i * e_j + f_i * f_j
                term2 = f_i * e_j - e_i * f_j
                vt_f = (e_i * e_i + f_i * f_i) - term1
                vt_r = (e_j * e_j + f_j * f_j) - term1
                gt2 = g_ij * term2
                bt2 = b_ij * term2
                mfP_v[sl] = g_ij * vt_f - bt2
                mfQ_v[sl] = -(b_ij * vt_f) - gt2
                mrP_v[sl] = g_ij * vt_r + bt2
                mrQ_v[sl] = -(b_ij * vt_r) + gt2
                m = (srcv == dst0) & (dstv == src0)
                flag_v[...] = flag_v[...] | m.astype(jnp.int32)
                return 0

            lax.fori_loop(0, GROUPS, group_body, 0, unroll=2)

        def scatter(b):
            src_v, dst_v, _, _, _ = ins[b]
            idxs = (src_v, src_v, dst_v, dst_v)
            descs = [pltpu.async_copy(m, a.at[i], sm, add=True)
                     for m, a, i, sm in zip(msgs, accs, idxs, ssems)]
            for d in descs:
                d.wait()

        issue_in(0, 0)
        issue_in(1, 1)
        wait_in(0, 0)
        issue_gathers(0)

        def outer(o, carry):
            for b in (0, 1):
                t = o * 2 + b
                nb = 1 - b
                wait_gathers(b)
                compute(b)

                @pl.when(t + 1 < CHUNKS)
                def _():
                    wait_in(t + 1, nb)
                    issue_gathers(nb)

                scatter(b)

                @pl.when(t + 2 < CHUNKS)
                def _():
                    issue_in(t + 2, b)
            return carry

        lax.fori_loop(0, CHUNKS // 2, outer, 0, unroll=False)
        plsc.subcore_barrier()

        out0 = c * N_PAD + row0
        for acc, out in zip(accs, (outfP_hbm, outfQ_hbm, outrP_hbm, outrQ_hbm)):
            pltpu.sync_copy(acc.at[pl.ds(row0, RPT)], stage_v)
            pltpu.sync_copy(stage_v, out.at[pl.ds(out0, RPT)])
        pltpu.sync_copy(flag_v, flags_hbm.at[pl.ds((c * NS + s) * L, L)])

    return body(edge_src, edge_dst, edge_g, edge_b, pred_e, pred_f,
                zeros_tile, probe, params)


def _tc_losses(planes, xymean, xystd):
    def body(pe_e, pe_f, ty_p, ty_q, ty_e, ty_f, mk_e, mk_f,
             af0P, af0Q, af1P, af1Q, ar0P, ar0Q, ar1P, ar1Q, bt, tvm,
             xym, xys,
             mse_num, mse_den, a_out, b_out, pv_num, pv_den):
        xys_v = xys[...]
        xym_v = xym[...]
        mk_e_v = mk_e[...]
        mk_f_v = mk_f[...]
        sq = (pe_e[...] - ty_e[...]) ** 2 * mk_e_v + (pe_f[...] - ty_f[...]) ** 2 * mk_f_v
        mse_num[...] = jnp.sum(sq).reshape(1, 1)
        mse_den[...] = (jnp.sum(mk_e_v) + jnp.sum(mk_f_v)).reshape(1, 1)

        u_p = ty_p[...] * (xys_v[0:1, 0:1] + 1e-7) + xym_v[0:1, 0:1] + af0P[...] + af1P[...]
        u_q = ty_q[...] * (xys_v[0:1, 1:2] + 1e-7) + xym_v[0:1, 1:2] + af0Q[...] + af1Q[...]
        v_p = ar0P[...] + ar1P[...]
        v_q = ar0Q[...] + ar1Q[...]
        a_out[...] = (jnp.sum(u_p * u_p) + jnp.sum(u_q * u_q)).reshape(1, 1)
        b_out[...] = (2.0 * (jnp.sum(u_p * v_p) + jnp.sum(u_q * v_q))
                      + jnp.sum(v_p * v_p) + jnp.sum(v_q * v_q)).reshape(1, 1)

        er = pe_e[...] * (xys_v[0:1, 2:3] + 1e-7) + xym_v[0:1, 2:3]
        fr = pe_f[...] * (xys_v[0:1, 3:4] + 1e-7) + xym_v[0:1, 3:4]
        vm_sq = er * er + fr * fr
        tv = tvm[...]
        ispv = (bt[...] == 1).astype(jnp.float32)
        pv_num[...] = jnp.sum(jnp.abs(vm_sq - tv * tv) * ispv).reshape(1, 1)
        pv_den[...] = jnp.sum(ispv).reshape(1, 1)

    scalar = jax.ShapeDtypeStruct((1, 1), jnp.float32)
    return pl.pallas_call(
        body,
        out_shape=(scalar,) * 6,
    )(*planes, xymean, xystd)


def kernel(pred_ef, target_y, mask, edge_index, edge_attr, bus_type, target_vm,
           xymean, xystd, edgemean, edgestd):
    pred_e = jnp.pad(pred_ef[:, 0], (0, N_PAD - N_NODES))
    pred_f = jnp.pad(pred_ef[:, 1], (0, N_PAD - N_NODES))
    zeros_tile = jnp.zeros((RPT,), jnp.float32)
    probe = jnp.broadcast_to(edge_index[:, 0:1], (2, L)).astype(jnp.int32)
    params = jnp.broadcast_to(
        jnp.stack([edgestd[0, 0] + 1e-7, edgestd[0, 1] + 1e-7,
                   edgemean[0, 0], edgemean[0, 1]])[:, None], (4, L))

    accfP, accfQ, accrP, accrQ, flags = _sc_edge_pass(
        edge_index[0].astype(jnp.int32), edge_index[1].astype(jnp.int32),
        edge_attr[:, 0], edge_attr[:, 1], pred_e, pred_f, zeros_tile, probe,
        params)

    def plane(x):
        return x.reshape(80, 1250)

    def halves(acc):
        return plane(acc[:N_NODES]), plane(acc[N_PAD:N_PAD + N_NODES])

    af0P, af1P = halves(accfP)
    af0Q, af1Q = halves(accfQ)
    ar0P, ar1P = halves(accrP)
    ar0Q, ar1Q = halves(accrQ)

    planes = (
        plane(pred_ef[:, 0]), plane(pred_ef[:, 1]),
        plane(target_y[:, 0]), plane(target_y[:, 1]),
        plane(target_y[:, 2]), plane(target_y[:, 3]),
        plane(mask[:, 2]), plane(mask[:, 3]),
        af0P, af0Q, af1P, af1Q, ar0P, ar0Q, ar1P, ar1Q,
        plane(bus_type.astype(jnp.int32)), plane(target_vm),
    )
    mse_num, mse_den, a_sum, b_sum, pv_num, pv_den = _tc_losses(
        planes, xymean, xystd)

    has_rev = jnp.any(flags != 0)
    w_rev = jnp.where(has_rev, 1.0, 0.0).astype(jnp.float32)

    loss_mse = mse_num[0, 0] / (mse_den[0, 0] + 1e-6)
    loss_phys = (a_sum[0, 0] + (1.0 - w_rev) * b_sum[0, 0]) / jnp.float32(N_NODES * 2)
    n_pv = pv_den[0, 0]
    loss_pv = jnp.where(n_pv > 0,
                        pv_num[0, 0] / jnp.maximum(n_pv, 1.0),
                        jnp.float32(0.0))
    total = 0.8 * loss_mse + 0.2 * loss_phys + 0.1 * loss_pv
    return (total, loss_mse, loss_phys, loss_pv)


# speedup vs baseline: 1.5628x; 1.0002x over previous
import functools

import jax
import jax.numpy as jnp
from jax import lax
from jax.experimental import pallas as pl
from jax.experimental.pallas import tpu as pltpu
from jax.experimental.pallas import tpu_sc as plsc

N_NODES = 100000
N_EDGES = 6400000
NC = 2
NS = 16
L = 16
N_PAD = 100096
RPT = N_PAD // NS
CHUNK = 4000
EDGES_PER_TILE = N_EDGES // (NC * NS)
CHUNKS = EDGES_PER_TILE // CHUNK
GROUPS = CHUNK // L


def _sc_edge_pass(edge_src, edge_dst, edge_g, edge_b, pred_e, pred_f,
                  zeros_tile, probe, params):
    mesh = plsc.VectorSubcoreMesh(
        core_axis_name="c", subcore_axis_name="s", num_cores=NC, num_subcores=NS
    )
    acc_ty = jax.ShapeDtypeStruct((NC * N_PAD,), jnp.float32)

    def buf(shape, dtype=jnp.float32):
        return pltpu.VMEM(shape, dtype)

    scratch = dict(
        stage_v=buf((RPT,)),
        flag_v=buf((L,), jnp.int32),
        probe_v=buf((2, L), jnp.int32),
        par_v=buf((4, L)),
        table_e=pltpu.VMEM_SHARED((N_PAD,), jnp.float32),
        table_f=pltpu.VMEM_SHARED((N_PAD,), jnp.float32),
        accfP=pltpu.VMEM_SHARED((N_PAD,), jnp.float32),
        accfQ=pltpu.VMEM_SHARED((N_PAD,), jnp.float32),
        accrP=pltpu.VMEM_SHARED((N_PAD,), jnp.float32),
        accrQ=pltpu.VMEM_SHARED((N_PAD,), jnp.float32),
        mfP_v=buf((CHUNK,)),
        mfQ_v=buf((CHUNK,)),
        mrP_v=buf((CHUNK,)),
        mrQ_v=buf((CHUNK,)),
        ssem0=pltpu.SemaphoreType.DMA,
        ssem1=pltpu.SemaphoreType.DMA,
        ssem2=pltpu.SemaphoreType.DMA,
        ssem3=pltpu.SemaphoreType.DMA,
    )
    for b in (0, 1):
        scratch.update({
            f"src_v{b}": buf((CHUNK,), jnp.int32),
            f"dst_v{b}": buf((CHUNK,), jnp.int32),
            f"gv_v{b}": buf((CHUNK,)),
            f"bv_v{b}": buf((CHUNK,)),
            f"es_v{b}": buf((CHUNK,)),
            f"fs_v{b}": buf((CHUNK,)),
            f"ed_v{b}": buf((CHUNK,)),
            f"fd_v{b}": buf((CHUNK,)),
            f"insem{b}": pltpu.SemaphoreType.DMA,
            f"gsem0{b}": pltpu.SemaphoreType.DMA,
            f"gsem1{b}": pltpu.SemaphoreType.DMA,
            f"gsem2{b}": pltpu.SemaphoreType.DMA,
            f"gsem3{b}": pltpu.SemaphoreType.DMA,
        })

    @functools.partial(
        pl.kernel,
        mesh=mesh,
        compiler_params=pltpu.CompilerParams(needs_layout_passes=False),
        out_type=(acc_ty, acc_ty, acc_ty, acc_ty,
                  jax.ShapeDtypeStruct((NC * NS * L,), jnp.int32)),
        scratch_types=scratch,
    )
    def body(es_hbm, ed_hbm, eg_hbm, eb_hbm, pe_hbm, pf_hbm, z_hbm, probe_hbm, par_hbm,
             outfP_hbm, outfQ_hbm, outrP_hbm, outrQ_hbm, flags_hbm,
             **refs):
        c = lax.axis_index("c")
        s = lax.axis_index("s")
        row0 = s * RPT
        stage_v = refs["stage_v"]
        flag_v = refs["flag_v"]
        probe_v = refs["probe_v"]
        par_v = refs["par_v"]
        table_e = refs["table_e"]
        table_f = refs["table_f"]
        accs = (refs["accfP"], refs["accfQ"], refs["accrP"], refs["accrQ"])
        msgs = (refs["mfP_v"], refs["mfQ_v"], refs["mrP_v"], refs["mrQ_v"])
        ssems = (refs["ssem0"], refs["ssem1"], refs["ssem2"], refs["ssem3"])
        ins = [(refs[f"src_v{b}"], refs[f"dst_v{b}"], refs[f"gv_v{b}"],
                refs[f"bv_v{b}"], refs[f"insem{b}"]) for b in (0, 1)]
        gbufs = [(refs[f"es_v{b}"], refs[f"fs_v{b}"], refs[f"ed_v{b}"],
                  refs[f"fd_v{b}"],
                  (refs[f"gsem0{b}"], refs[f"gsem1{b}"],
                   refs[f"gsem2{b}"], refs[f"gsem3{b}"])) for b in (0, 1)]

        pltpu.sync_copy(pe_hbm.at[pl.ds(row0, RPT)], stage_v)
        pltpu.sync_copy(stage_v, table_e.at[pl.ds(row0, RPT)])
        pltpu.sync_copy(pf_hbm.at[pl.ds(row0, RPT)], stage_v)
        pltpu.sync_copy(stage_v, table_f.at[pl.ds(row0, RPT)])
        pltpu.sync_copy(z_hbm, stage_v)
        for acc in accs:
            pltpu.sync_copy(stage_v, acc.at[pl.ds(row0, RPT)])
        pltpu.sync_copy(probe_hbm, probe_v)
        pltpu.sync_copy(par_hbm, par_v)
        flag_v[...] = jnp.zeros((L,), jnp.int32)
        plsc.subcore_barrier()

        src0 = probe_v[0, :]
        dst0 = probe_v[1, :]
        esg = par_v[0, :]
        esb = par_v[1, :]
        emg = par_v[2, :]
        emb = par_v[3, :]
        tile_base = (c * NS + s) * EDGES_PER_TILE

        def issue_in(t, b):
            base = tile_base + t * CHUNK
            src_v, dst_v, gv_v, bv_v, insem = ins[b]
            pltpu.async_copy(es_hbm.at[pl.ds(base, CHUNK)], src_v, insem)
            pltpu.async_copy(ed_hbm.at[pl.ds(base, CHUNK)], dst_v, insem)
            pltpu.async_copy(eg_hbm.at[pl.ds(base, CHUNK)], gv_v, insem)
            pltpu.async_copy(eb_hbm.at[pl.ds(base, CHUNK)], bv_v, insem)

        def wait_in(t, b):
            base = tile_base + t * CHUNK
            src_v, dst_v, gv_v, bv_v, insem = ins[b]
            pltpu.make_async_copy(es_hbm.at[pl.ds(base, CHUNK)], src_v, insem).wait()
            pltpu.make_async_copy(ed_hbm.at[pl.ds(base, CHUNK)], dst_v, insem).wait()
            pltpu.make_async_copy(eg_hbm.at[pl.ds(base, CHUNK)], gv_v, insem).wait()
            pltpu.make_async_copy(eb_hbm.at[pl.ds(base, CHUNK)], bv_v, insem).wait()

        def issue_gathers(b):
            src_v, dst_v, _, _, _ = ins[b]
            es_v, fs_v, ed_v, fd_v, gsems = gbufs[b]
            pltpu.async_copy(table_e.at[src_v], es_v, gsems[0])
            pltpu.async_copy(table_f.at[src_v], fs_v, gsems[1])
            pltpu.async_copy(table_e.at[dst_v], ed_v, gsems[2])
            pltpu.async_copy(table_f.at[dst_v], fd_v, gsems[3])

        def wait_gathers(b):
            src_v, dst_v, _, _, _ = ins[b]
            es_v, fs_v, ed_v, fd_v, gsems = gbufs[b]
            pltpu.make_async_copy(table_e.at[src_v], es_v, gsems[0]).wait()
            pltpu.make_async_copy(table_f.at[src_v], fs_v, gsems[1]).wait()
            pltpu.make_async_copy(table_e.at[dst_v], ed_v, gsems[2]).wait()
            pltpu.make_async_copy(table_f.at[dst_v], fd_v, gsems[3]).wait()

        def compute(b):
            src_v, dst_v, gv_v, bv_v, _ = ins[b]
            es_v, fs_v, ed_v, fd_v, _ = gbufs[b]
            mfP_v, mfQ_v, mrP_v, mrQ_v = msgs

            def group_body(g, _):
                sl = pl.ds(g * L, L)
                srcv = src_v[sl]
                dstv = dst_v[sl]
                e_i = es_v[sl]
                f_i = fs_v[sl]
                e_j = ed_v[sl]
                f_j = fd_v[sl]
                g_ij = gv_v[sl] * esg + emg
                b_ij = bv_v[sl] * esb + emb
                term1 = e_i * e_j + f_i * f_j
                term2 = f_i * e_j - e_i * f_j
                vt_f = (e_i * e_i + f_i * f_i) - term1
                vt_r = (e_j * e_j + f_j * f_j) - term1
                gt2 = g_ij * term2
                bt2 = b_ij * term2
                mfP_v[sl] = g_ij * vt_f - bt2
                mfQ_v[sl] = -(b_ij * vt_f) - gt2
                mrP_v[sl] = g_ij * vt_r + bt2
                mrQ_v[sl] = -(b_ij * vt_r) + gt2
                m = (srcv == dst0) & (dstv == src0)
                flag_v[...] = flag_v[...] | m.astype(jnp.int32)
                return 0

            lax.fori_loop(0, GROUPS, group_body, 0, unroll=False)

        def scatter(b):
            src_v, dst_v, _, _, _ = ins[b]
            idxs = (src_v, src_v, dst_v, dst_v)
            descs = [pltpu.async_copy(m, a.at[i], sm, add=True)
                     for m, a, i, sm in zip(msgs, accs, idxs, ssems)]
            for d in descs:
                d.wait()

        issue_in(0, 0)
        issue_in(1, 1)
        wait_in(0, 0)
        issue_gathers(0)

        def outer(o, carry):
            for b in (0, 1):
                t = o * 2 + b
                nb = 1 - b
                wait_gathers(b)
                compute(b)

                @pl.when(t + 1 < CHUNKS)
                def _():
                    wait_in(t + 1, nb)
                    issue_gathers(nb)

                scatter(b)

                @pl.when(t + 2 < CHUNKS)
                def _():
                    issue_in(t + 2, b)
            return carry

        lax.fori_loop(0, CHUNKS // 2, outer, 0, unroll=False)
        plsc.subcore_barrier()

        out0 = c * N_PAD + row0
        for acc, out in zip(accs, (outfP_hbm, outfQ_hbm, outrP_hbm, outrQ_hbm)):
            pltpu.sync_copy(acc.at[pl.ds(row0, RPT)], stage_v)
            pltpu.sync_copy(stage_v, out.at[pl.ds(out0, RPT)])
        pltpu.sync_copy(flag_v, flags_hbm.at[pl.ds((c * NS + s) * L, L)])

    return body(edge_src, edge_dst, edge_g, edge_b, pred_e, pred_f,
                zeros_tile, probe, params)


def _tc_losses(planes, xymean, xystd):
    def body(pe_e, pe_f, ty_p, ty_q, ty_e, ty_f, mk_e, mk_f,
             af0P, af0Q, af1P, af1Q, ar0P, ar0Q, ar1P, ar1Q, bt, tvm,
             xym, xys,
             mse_num, mse_den, a_out, b_out, pv_num, pv_den):
        xys_v = xys[...]
        xym_v = xym[...]
        mk_e_v = mk_e[...]
        mk_f_v = mk_f[...]
        sq = (pe_e[...] - ty_e[...]) ** 2 * mk_e_v + (pe_f[...] - ty_f[...]) ** 2 * mk_f_v
        mse_num[...] = jnp.sum(sq).reshape(1, 1)
        mse_den[...] = (jnp.sum(mk_e_v) + jnp.sum(mk_f_v)).reshape(1, 1)

        u_p = ty_p[...] * (xys_v[0:1, 0:1] + 1e-7) + xym_v[0:1, 0:1] + af0P[...] + af1P[...]
        u_q = ty_q[...] * (xys_v[0:1, 1:2] + 1e-7) + xym_v[0:1, 1:2] + af0Q[...] + af1Q[...]
        v_p = ar0P[...] + ar1P[...]
        v_q = ar0Q[...] + ar1Q[...]
        a_out[...] = (jnp.sum(u_p * u_p) + jnp.sum(u_q * u_q)).reshape(1, 1)
        b_out[...] = (2.0 * (jnp.sum(u_p * v_p) + jnp.sum(u_q * v_q))
                      + jnp.sum(v_p * v_p) + jnp.sum(v_q * v_q)).reshape(1, 1)

        er = pe_e[...] * (xys_v[0:1, 2:3] + 1e-7) + xym_v[0:1, 2:3]
        fr = pe_f[...] * (xys_v[0:1, 3:4] + 1e-7) + xym_v[0:1, 3:4]
        vm_sq = er * er + fr * fr
        tv = tvm[...]
        ispv = (bt[...] == 1).astype(jnp.float32)
        pv_num[...] = jnp.sum(jnp.abs(vm_sq - tv * tv) * ispv).reshape(1, 1)
        pv_den[...] = jnp.sum(ispv).reshape(1, 1)

    scalar = jax.ShapeDtypeStruct((1, 1), jnp.float32)
    return pl.pallas_call(
        body,
        out_shape=(scalar,) * 6,
    )(*planes, xymean, xystd)


def kernel(pred_ef, target_y, mask, edge_index, edge_attr, bus_type, target_vm,
           xymean, xystd, edgemean, edgestd):
    pred_e = jnp.pad(pred_ef[:, 0], (0, N_PAD - N_NODES))
    pred_f = jnp.pad(pred_ef[:, 1], (0, N_PAD - N_NODES))
    zeros_tile = jnp.zeros((RPT,), jnp.float32)
    probe = jnp.broadcast_to(edge_index[:, 0:1], (2, L)).astype(jnp.int32)
    params = jnp.broadcast_to(
        jnp.stack([edgestd[0, 0] + 1e-7, edgestd[0, 1] + 1e-7,
                   edgemean[0, 0], edgemean[0, 1]])[:, None], (4, L))

    accfP, accfQ, accrP, accrQ, flags = _sc_edge_pass(
        edge_index[0].astype(jnp.int32), edge_index[1].astype(jnp.int32),
        edge_attr[:, 0], edge_attr[:, 1], pred_e, pred_f, zeros_tile, probe,
        params)

    def plane(x):
        return x.reshape(80, 1250)

    def halves(acc):
        return plane(acc[:N_NODES]), plane(acc[N_PAD:N_PAD + N_NODES])

    af0P, af1P = halves(accfP)
    af0Q, af1Q = halves(accfQ)
    ar0P, ar1P = halves(accrP)
    ar0Q, ar1Q = halves(accrQ)

    planes = (
        plane(pred_ef[:, 0]), plane(pred_ef[:, 1]),
        plane(target_y[:, 0]), plane(target_y[:, 1]),
        plane(target_y[:, 2]), plane(target_y[:, 3]),
        plane(mask[:, 2]), plane(mask[:, 3]),
        af0P, af0Q, af1P, af1Q, ar0P, ar0Q, ar1P, ar1Q,
        plane(bus_type.astype(jnp.int32)), plane(target_vm),
    )
    mse_num, mse_den, a_sum, b_sum, pv_num, pv_den = _tc_losses(
        planes, xymean, xystd)

    has_rev = jnp.any(flags != 0)
    w_rev = jnp.where(has_rev, 1.0, 0.0).astype(jnp.float32)

    loss_mse = mse_num[0, 0] / (mse_den[0, 0] + 1e-6)
    loss_phys = (a_sum[0, 0] + (1.0 - w_rev) * b_sum[0, 0]) / jnp.float32(N_NODES * 2)
    n_pv = pv_den[0, 0]
    loss_pv = jnp.where(n_pv > 0,
                        pv_num[0, 0] / jnp.maximum(n_pv, 1.0),
                        jnp.float32(0.0))
    total = 0.8 * loss_mse + 0.2 * loss_phys + 0.1 * loss_pv
    return (total, loss_mse, loss_phys, loss_pv)
